# Initial kernel scaffold; baseline (speedup 1.0000x reference)
import jax
import jax.numpy as jnp
from jax.experimental import pallas as pl


def kernel(x):
    raise NotImplementedError("write your pallas kernel here")


# speedup vs baseline: 22.8958x; 22.8958x over previous
import functools

import jax
import jax.numpy as jnp
from jax import lax
from jax.experimental import pallas as pl
from jax.experimental.pallas import tpu as pltpu
from jax.experimental.pallas import tpu_sc as plsc

_PERCENTILE = 99.0 / 100.0
_GAMMA = 0.95
_Q_MAX = float(2 ** (8 - 1) - 1) * 2.0
_INIT_ACT_MIN = -100.0
_INIT_ACT_MAX = 100.0

_N = 4096 * 4096
_K_LO = round((1.0 - _PERCENTILE) * _N) - 1
_K_HI = round(_PERCENTILE * _N) - 1

_NC = 2
_NS = 16
_L = 16
_NW = _NC * _NS

_BINS = 65536
_CHUNK = 16384

_MIN32 = -(2 ** 31)


def _key16(xv):
    xi = lax.bitcast_convert_type(xv, jnp.int32)
    m = lax.shift_right_arithmetic(xi, 31)
    return lax.bitwise_xor(xi, lax.bitwise_or(m, jnp.int32(_MIN32)))


_sc_mesh = plsc.VectorSubcoreMesh(
    core_axis_name="c", subcore_axis_name="s", num_cores=_NC, num_subcores=_NS
)


@functools.partial(
    pl.kernel,
    out_type=jax.ShapeDtypeStruct((_NW, _BINS), jnp.int32),
    mesh=_sc_mesh,
    scratch_types=[
        pltpu.VMEM((_CHUNK,), jnp.float32),
        pltpu.VMEM((_BINS,), jnp.int32),
    ],
    compiler_params=pltpu.CompilerParams(needs_layout_passes=False),
)
def _sc_hist_hi(x_hbm, out_hbm, buf, hist):
    wid = lax.axis_index("s") * _NC + lax.axis_index("c")
    per_w = _N // _NW
    base = wid * per_w

    zeros = jnp.zeros((_L,), jnp.int32)

    def zero_body(i, carry):
        hist[pl.ds(i * _L, _L)] = zeros
        return carry

    lax.fori_loop(0, _BINS // _L, zero_body, 0)

    ones = jnp.ones((_L,), jnp.int32)

    def chunk_body(ci, carry):
        pltpu.sync_copy(x_hbm.at[pl.ds(base + ci * _CHUNK, _CHUNK)], buf)

        def vec_body(j, c2):
            xv = buf[pl.ds(j * _L, _L)]
            bins = lax.shift_right_logical(_key16(xv), 16)
            plsc.addupdate_scatter(hist, [bins], ones)
            return c2

        lax.fori_loop(0, _CHUNK // _L, vec_body, 0)
        return carry

    lax.fori_loop(0, per_w // _CHUNK, chunk_body, 0)
    pltpu.sync_copy(hist, out_hbm.at[wid])


@functools.partial(
    pl.kernel,
    out_type=jax.ShapeDtypeStruct((_NC, _NS, _BINS), jnp.int32),
    mesh=_sc_mesh,
    scratch_types=[
        pltpu.VMEM((_CHUNK,), jnp.float32),
        pltpu.VMEM((_BINS,), jnp.int32),
        pltpu.VMEM((8, _L), jnp.int32),
    ],
    compiler_params=pltpu.CompilerParams(needs_layout_passes=False),
)
def _sc_hist_lo(x_hbm, sel_hbm, out_hbm, buf, hist, sel_v):
    c = lax.axis_index("c")
    s = lax.axis_index("s")
    per_s = _N // _NS
    base = s * per_s

    pltpu.sync_copy(sel_hbm, sel_v)
    b_cand = jnp.where(jnp.broadcast_to(c == 0, (_L,)), sel_v[0, :], sel_v[1, :])

    zeros = jnp.zeros((_L,), jnp.int32)

    def zero_body(i, carry):
        hist[pl.ds(i * _L, _L)] = zeros
        return carry

    lax.fori_loop(0, _BINS // _L, zero_body, 0)

    ones = jnp.ones((_L,), jnp.int32)
    lo_mask = jnp.int32(0xFFFF)

    def chunk_body(ci, carry):
        pltpu.sync_copy(x_hbm.at[pl.ds(base + ci * _CHUNK, _CHUNK)], buf)

        def vec_body(j, c2):
            xv = buf[pl.ds(j * _L, _L)]
            key = _key16(xv)
            hi = lax.shift_right_logical(key, 16)
            lo = lax.bitwise_and(key, lo_mask)
            msk = hi == b_cand
            plsc.addupdate_scatter(
                hist, [jnp.where(msk, lo, 0)], ones, mask=msk
            )
            return c2

        lax.fori_loop(0, _CHUNK // _L, vec_body, 0)
        return carry

    lax.fori_loop(0, per_s // _CHUNK, chunk_body, 0)
    pltpu.sync_copy(hist, out_hbm.at[c, s])


def _cumsum_lanes(v):
    n = v.shape[-1]
    lane = lax.broadcasted_iota(jnp.int32, v.shape, 1)
    shift = 1
    while shift < n:
        rolled = pltpu.roll(v, shift, 1)
        v = v + jnp.where(lane >= shift, rolled, 0)
        shift *= 2
    return v


def _tc_select_hi_body(h_ref, o_ref):
    h = h_ref[...]
    tot = jnp.sum(h, axis=0, keepdims=True)
    cum = _cumsum_lanes(tot)

    t_lo = jnp.int32(_K_LO + 1)
    t_hi = jnp.int32(_K_HI + 1)
    below_lo = cum < t_lo
    below_hi = cum < t_hi
    b_lo = jnp.sum(jnp.where(below_lo, 1, 0), axis=1, keepdims=True)
    b_hi = jnp.sum(jnp.where(below_hi, 1, 0), axis=1, keepdims=True)
    cb_lo = jnp.max(jnp.where(below_lo, cum, 0), axis=1, keepdims=True)
    cb_hi = jnp.max(jnp.where(below_hi, cum, 0), axis=1, keepdims=True)

    row = lax.broadcasted_iota(jnp.int32, (8, 128), 0)
    out = jnp.where(
        row == 0,
        b_lo,
        jnp.where(row == 1, b_hi, jnp.where(row == 2, cb_lo, cb_hi)),
    )
    o_ref[...] = out


_tc_select_hi = pl.pallas_call(
    _tc_select_hi_body,
    out_shape=jax.ShapeDtypeStruct((8, 128), jnp.int32),
)


def _tc_select_lo_body(h_ref, sel_ref, o_ref):
    h0 = jnp.sum(h_ref[0], axis=0, keepdims=True)
    h1 = jnp.sum(h_ref[1], axis=0, keepdims=True)
    cum0 = _cumsum_lanes(h0)
    cum1 = _cumsum_lanes(h1)

    b_lo = sel_ref[0:1, 0:1]
    b_hi = sel_ref[1:2, 0:1]
    cb_lo = sel_ref[2:3, 0:1]
    cb_hi = sel_ref[3:4, 0:1]

    t0 = jnp.int32(_K_LO + 1) - cb_lo
    t1 = jnp.int32(_K_HI + 1) - cb_hi
    l0 = jnp.sum(jnp.where(cum0 < t0, 1, 0), axis=1, keepdims=True)
    l1 = jnp.sum(jnp.where(cum1 < t1, 1, 0), axis=1, keepdims=True)

    k0 = lax.bitwise_or(lax.shift_left(b_lo, 16), l0)
    k1 = lax.bitwise_or(lax.shift_left(b_hi, 16), l1)

    def _unmap(k):
        bits = jnp.where(
            k < 0, lax.bitwise_xor(k, jnp.int32(_MIN32)), lax.bitwise_not(k)
        )
        return lax.bitcast_convert_type(bits, jnp.float32)

    q_lo = _unmap(k0)
    q_hi = _unmap(k1)
    act_min = jnp.float32(_INIT_ACT_MIN * _GAMMA) + q_lo * jnp.float32(1.0 - _GAMMA)
    act_max = jnp.float32(_INIT_ACT_MAX * _GAMMA) + q_hi * jnp.float32(1.0 - _GAMMA)

    row = lax.broadcasted_iota(jnp.int32, (8, 128), 0)
    o_ref[...] = jnp.where(row == 0, act_min, jnp.where(row == 1, act_max, 0.0))


_tc_select_lo = pl.pallas_call(
    _tc_select_lo_body,
    out_shape=jax.ShapeDtypeStruct((8, 128), jnp.float32),
)


_QROWS = 256


def _tc_quantize_body(x_ref, p_ref, o_ref):
    act_min = p_ref[0:1, 0:1]
    act_max = p_ref[1:2, 0:1]
    rng = act_max - act_min
    x = x_ref[...]
    y = jnp.round((x - act_min) * (jnp.float32(_Q_MAX) / rng))
    y = y * (rng / jnp.float32(_Q_MAX)) + act_min
    clamp_min = act_min - rng * jnp.float32(0.5 / _Q_MAX)
    clamp_max = act_max + rng * jnp.float32(0.5 / _Q_MAX)
    o_ref[...] = jnp.clip(y, clamp_min, clamp_max)


_tc_quantize = pl.pallas_call(
    _tc_quantize_body,
    grid=(4096 // _QROWS,),
    in_specs=[
        pl.BlockSpec((_QROWS, 4096), lambda i: (i, 0)),
        pl.BlockSpec((8, 128), lambda i: (0, 0)),
    ],
    out_specs=pl.BlockSpec((_QROWS, 4096), lambda i: (i, 0)),
    out_shape=jax.ShapeDtypeStruct((4096, 4096), jnp.float32),
)


def kernel(x):
    xf = x.reshape(-1)
    hist_hi = _sc_hist_hi(xf)
    sel = _tc_select_hi(hist_hi)
    hist_lo = _sc_hist_lo(xf, sel[:, :_L])
    params = _tc_select_lo(hist_lo, sel)
    return _tc_quantize(x, params)
